# Initial kernel scaffold; baseline (speedup 1.0000x reference)
import jax
import jax.numpy as jnp
from jax.experimental import pallas as pl


def kernel(edge_index, feat, pos_undirected, seed, deg_table, W1, b1, W2, b2, P0_W, P0_b, P1_W, P1_b):
    raise NotImplementedError("write your pallas kernel here")


# speedup vs baseline: 7.8312x; 7.8312x over previous
import functools

import jax
import jax.numpy as jnp
from jax import lax
from jax.experimental import pallas as pl
from jax.experimental.pallas import tpu as pltpu
from jax.experimental.pallas import tpu_sc as plsc

N = 10000
E = 320000
NC = 2
NS = 16
EPT = E // (NC * NS)
SUB = 80
GRP = 400
NGRP = EPT // GRP
HID = 64


def _deg_body(src, deg2, deg_sh, idx_v, ones_v, zbuf):
    c = lax.axis_index("c")
    s = lax.axis_index("s")
    for j in range(5):
        ones_v[pl.ds(j * 16, 16)] = jnp.ones((16,), jnp.int32)
    for j in range(64):
        zbuf[pl.ds(j * 16, 16)] = jnp.zeros((16,), jnp.int32)

    @pl.when(s < 10)
    def _():
        pltpu.sync_copy(zbuf.at[pl.ds(0, 1000)], deg_sh.at[pl.ds(s * 1000, 1000)])

    plsc.subcore_barrier()

    ebase = (c * NS + s) * EPT

    def grp(g, carry):
        gb = ebase + g * GRP
        for j in range(5):
            pltpu.sync_copy(src.at[pl.ds(gb + j * SUB, SUB)], idx_v.at[j])
        for j in range(5):
            pltpu.sync_copy(ones_v, deg_sh.at[idx_v.at[j]], add=True)
        return carry

    lax.fori_loop(0, NGRP, grp, 0)
    plsc.subcore_barrier()

    @pl.when(s < 10)
    def _():
        pltpu.sync_copy(deg_sh.at[pl.ds(s * 1000, 1000)], zbuf.at[pl.ds(0, 1000)])
        pltpu.sync_copy(zbuf.at[pl.ds(0, 1000)],
                        deg2.at[pl.ds(c * N + s * 1000, 1000)])


@functools.cache
def _make_deg_kernel():
    return pl.kernel(
        _deg_body,
        out_type=jax.ShapeDtypeStruct((NC * N,), jnp.int32),
        mesh=plsc.VectorSubcoreMesh(core_axis_name="c", subcore_axis_name="s"),
        scratch_types=[
            pltpu.VMEM_SHARED((N,), jnp.int32),
            pltpu.VMEM((5, SUB), jnp.int32),
            pltpu.VMEM((SUB,), jnp.int32),
            pltpu.VMEM((1024,), jnp.int32),
        ],
        compiler_params=pltpu.CompilerParams(use_tc_tiling_on_sc=False),
    )


def _agg_body(src, dst, y, out, agg_sh, src_v, dst_v, rows_v, sem):
    c = lax.axis_index("c")
    s = lax.axis_index("s")

    def zrow(i, carry):
        for j in range(4):
            rows_v[i, pl.ds(j * 16, 16)] = jnp.zeros((16,), jnp.float32)
        return carry

    lax.fori_loop(0, GRP, zrow, 0)

    @pl.when(s < 10)
    def _():
        base = s * 1000
        pltpu.sync_copy(rows_v.at[pl.ds(0, 400)], agg_sh.at[pl.ds(base, 400)])
        pltpu.sync_copy(rows_v.at[pl.ds(0, 400)], agg_sh.at[pl.ds(base + 400, 400)])
        pltpu.sync_copy(rows_v.at[pl.ds(0, 200)], agg_sh.at[pl.ds(base + 800, 200)])

    plsc.subcore_barrier()

    ebase = (c * NS + s) * EPT

    def grp(g, carry):
        gb = ebase + g * GRP
        pltpu.sync_copy(src.at[pl.ds(gb, GRP)], src_v)
        for j in range(5):
            pltpu.sync_copy(dst.at[pl.ds(gb + j * SUB, SUB)], dst_v.at[j])
        cps = [
            pltpu.async_copy(y.at[src_v.at[pl.ds(j * SUB, SUB)]],
                             rows_v.at[pl.ds(j * SUB, SUB)], sem)
            for j in range(5)
        ]
        for cp in cps:
            cp.wait()
        for j in range(5):
            pltpu.sync_copy(rows_v.at[pl.ds(j * SUB, SUB)],
                            agg_sh.at[dst_v.at[j]], add=True)
        return carry

    lax.fori_loop(0, NGRP, grp, 0)
    plsc.subcore_barrier()

    @pl.when(s < 10)
    def _():
        for off, nrows in ((0, 400), (400, 400), (800, 200)):
            pltpu.sync_copy(agg_sh.at[pl.ds(s * 1000 + off, nrows)],
                            rows_v.at[pl.ds(0, nrows)])
            pltpu.sync_copy(rows_v.at[pl.ds(0, nrows)],
                            out.at[pl.ds(c * N + s * 1000 + off, nrows), :])


@functools.cache
def _make_agg_kernel():
    return pl.kernel(
        _agg_body,
        out_type=jax.ShapeDtypeStruct((NC * N, HID), jnp.float32),
        mesh=plsc.VectorSubcoreMesh(core_axis_name="c", subcore_axis_name="s"),
        scratch_types=[
            pltpu.VMEM_SHARED((N, HID), jnp.float32),
            pltpu.VMEM((GRP,), jnp.int32),
            pltpu.VMEM((5, SUB), jnp.int32),
            pltpu.VMEM((GRP, HID), jnp.float32),
            pltpu.SemaphoreType.DMA,
        ],
        compiler_params=pltpu.CompilerParams(use_tc_tiling_on_sc=False),
    )


def _feat_body(deg2, pos, seedf, feat, dt, w1a, w1b, w1s, w1c,
               p0a, p0b, p0s, p0c, p0bias, y_out, score0_out):
    deg = deg2[0, :] + deg2[1, :]
    degc = jnp.clip(deg, 0, dt.shape[0] - 1)
    oh = (degc[:, None] == lax.broadcasted_iota(jnp.int32, (1, dt.shape[0]), 1)
          ).astype(jnp.float32)
    demb = jnp.dot(oh, dt[...], preferred_element_type=jnp.float32)
    y = (jnp.dot(pos[...], w1a[...], preferred_element_type=jnp.float32)
         + jnp.dot(demb, w1b[...], preferred_element_type=jnp.float32)
         + seedf[...] * w1s[...]
         + jnp.dot(feat[...], w1c[...], preferred_element_type=jnp.float32))
    y_out[...] = y
    s_pos = jnp.sum(pos[...], axis=0, keepdims=True)
    s_demb = jnp.sum(demb, axis=0, keepdims=True)
    s_seed = jnp.sum(seedf[...], axis=0, keepdims=True)
    s_feat = jnp.sum(feat[...], axis=0, keepdims=True)
    score0_out[...] = (
        jnp.dot(s_pos, p0a[...], preferred_element_type=jnp.float32)
        + jnp.dot(s_demb, p0b[...], preferred_element_type=jnp.float32)
        + s_seed * p0s[...]
        + jnp.dot(s_feat, p0c[...], preferred_element_type=jnp.float32)
        + p0bias[...])


def _finish_body(y, agg2, b1, w2, b2, p1w, p1b, score0, out):
    h = jnp.maximum(y[...] + agg2[0] + agg2[1] + b1[...], 0.0)
    h2 = jnp.maximum(
        jnp.dot(h, w2[...], preferred_element_type=jnp.float32) + b2[...], 0.0)
    pooled1 = jnp.sum(h2, axis=0, keepdims=True)
    out[...] = (score0[...]
                + jnp.dot(pooled1, p1w[...], preferred_element_type=jnp.float32)
                + p1b[...])


@jax.jit
def kernel(edge_index, feat, pos_undirected, seed, deg_table,
           W1, b1, W2, b2, P0_W, P0_b, P1_W, P1_b):
    pos_w = pos_undirected.shape[1]
    demb_w = deg_table.shape[1]
    seedf = seed.astype(jnp.float32)[:, None]

    w1a = W1[:pos_w]
    w1b = W1[pos_w:pos_w + demb_w]
    w1s = W1[pos_w + demb_w:pos_w + demb_w + 1]
    w1c = W1[pos_w + demb_w + 1:]
    p0a = P0_W[:pos_w]
    p0b = P0_W[pos_w:pos_w + demb_w]
    p0s = P0_W[pos_w + demb_w:pos_w + demb_w + 1]
    p0c = P0_W[pos_w + demb_w + 1:]

    src_idx = edge_index[0]
    dst_idx = edge_index[1]
    deg2 = _make_deg_kernel()(src_idx).reshape(NC, N)

    y, score0 = pl.pallas_call(
        _feat_body,
        out_shape=[
            jax.ShapeDtypeStruct((N, HID), jnp.float32),
            jax.ShapeDtypeStruct((1, HID), jnp.float32),
        ],
    )(deg2, pos_undirected, seedf, feat, deg_table,
      w1a, w1b, w1s, w1c, p0a, p0b, p0s, p0c, P0_b[None, :])

    agg2 = _make_agg_kernel()(src_idx, dst_idx, y).reshape(NC, N, HID)

    score = pl.pallas_call(
        _finish_body,
        out_shape=jax.ShapeDtypeStruct((1, HID), jnp.float32),
    )(y, agg2, b1[None, :], W2, b2[None, :], P1_W, P1_b[None, :], score0)
    return score


# speedup vs baseline: 8.9762x; 1.1462x over previous
import functools

import jax
import jax.numpy as jnp
from jax import lax
from jax.experimental import pallas as pl
from jax.experimental.pallas import tpu as pltpu
from jax.experimental.pallas import tpu_sc as plsc

N = 10000
E = 320000
NC = 2
NS = 16
EPT = E // (NC * NS)
SUB = 80
GRP = 400
NGRP = EPT // GRP
HID = 64


def _deg_body(src, deg2, deg_sh, idx_v, ones_v, zbuf):
    c = lax.axis_index("c")
    s = lax.axis_index("s")
    for j in range(5):
        ones_v[pl.ds(j * 16, 16)] = jnp.ones((16,), jnp.int32)
    for j in range(64):
        zbuf[pl.ds(j * 16, 16)] = jnp.zeros((16,), jnp.int32)

    @pl.when(s < 10)
    def _():
        pltpu.sync_copy(zbuf.at[pl.ds(0, 1000)], deg_sh.at[pl.ds(s * 1000, 1000)])

    plsc.subcore_barrier()

    ebase = (c * NS + s) * EPT

    def grp(g, carry):
        gb = ebase + g * GRP
        for j in range(5):
            pltpu.sync_copy(src.at[pl.ds(gb + j * SUB, SUB)], idx_v.at[j])
        for j in range(5):
            pltpu.sync_copy(ones_v, deg_sh.at[idx_v.at[j]], add=True)
        return carry

    lax.fori_loop(0, NGRP, grp, 0)
    plsc.subcore_barrier()

    @pl.when(s < 10)
    def _():
        pltpu.sync_copy(deg_sh.at[pl.ds(s * 1000, 1000)], zbuf.at[pl.ds(0, 1000)])
        pltpu.sync_copy(zbuf.at[pl.ds(0, 1000)],
                        deg2.at[pl.ds(c * N + s * 1000, 1000)])


@functools.cache
def _make_deg_kernel():
    return pl.kernel(
        _deg_body,
        out_type=jax.ShapeDtypeStruct((NC * N,), jnp.int32),
        mesh=plsc.VectorSubcoreMesh(core_axis_name="c", subcore_axis_name="s"),
        scratch_types=[
            pltpu.VMEM_SHARED((N,), jnp.int32),
            pltpu.VMEM((5, SUB), jnp.int32),
            pltpu.VMEM((SUB,), jnp.int32),
            pltpu.VMEM((1024,), jnp.int32),
        ],
        compiler_params=pltpu.CompilerParams(use_tc_tiling_on_sc=False),
    )


def _agg_body(src, dst, y, out, agg_sh, src_v, dst_v, rows_v, sem0, sem1):
    c = lax.axis_index("c")
    s = lax.axis_index("s")

    def zrow(i, carry):
        for j in range(4):
            rows_v[i, pl.ds(j * 16, 16)] = jnp.zeros((16,), jnp.float32)
        return carry

    lax.fori_loop(0, GRP, zrow, 0)

    @pl.when(s < 10)
    def _():
        base = s * 1000
        pltpu.sync_copy(rows_v.at[pl.ds(0, 400)], agg_sh.at[pl.ds(base, 400)])
        pltpu.sync_copy(rows_v.at[pl.ds(0, 400)], agg_sh.at[pl.ds(base + 400, 400)])
        pltpu.sync_copy(rows_v.at[pl.ds(0, 200)], agg_sh.at[pl.ds(base + 800, 200)])

    plsc.subcore_barrier()

    ebase = (c * NS + s) * EPT
    sems = (sem0, sem1)

    def load_idx(g, b):
        gb = ebase + g * GRP
        pltpu.sync_copy(src.at[pl.ds(gb, GRP)], src_v.at[pl.ds(b * GRP, GRP)])
        for j in range(5):
            pltpu.sync_copy(dst.at[pl.ds(gb + j * SUB, SUB)],
                            dst_v.at[b * 5 + j])

    def issue_gathers(b):
        return [
            pltpu.async_copy(
                y.at[src_v.at[pl.ds(b * GRP + j * SUB, SUB)]],
                rows_v.at[pl.ds(b * GRP + j * SUB, SUB)], sems[b])
            for j in range(5)
        ]

    load_idx(0, 0)
    pend = issue_gathers(0)
    for g in range(NGRP):
        b = g & 1
        if g + 1 < NGRP:
            load_idx(g + 1, 1 - b)
            nxt = issue_gathers(1 - b)
        for cp in pend:
            cp.wait()
        for j in range(5):
            pltpu.sync_copy(rows_v.at[pl.ds(b * GRP + j * SUB, SUB)],
                            agg_sh.at[dst_v.at[b * 5 + j]], add=True)
        if g + 1 < NGRP:
            pend = nxt
    plsc.subcore_barrier()

    @pl.when(s < 10)
    def _():
        for off, nrows in ((0, 400), (400, 400), (800, 200)):
            pltpu.sync_copy(agg_sh.at[pl.ds(s * 1000 + off, nrows)],
                            rows_v.at[pl.ds(0, nrows)])
            pltpu.sync_copy(rows_v.at[pl.ds(0, nrows)],
                            out.at[pl.ds(c * N + s * 1000 + off, nrows), :])


@functools.cache
def _make_agg_kernel():
    return pl.kernel(
        _agg_body,
        out_type=jax.ShapeDtypeStruct((NC * N, HID), jnp.float32),
        mesh=plsc.VectorSubcoreMesh(core_axis_name="c", subcore_axis_name="s"),
        scratch_types=[
            pltpu.VMEM_SHARED((N, HID), jnp.float32),
            pltpu.VMEM((2 * GRP,), jnp.int32),
            pltpu.VMEM((10, SUB), jnp.int32),
            pltpu.VMEM((2 * GRP, HID), jnp.float32),
            pltpu.SemaphoreType.DMA,
            pltpu.SemaphoreType.DMA,
        ],
        compiler_params=pltpu.CompilerParams(use_tc_tiling_on_sc=False),
    )


def _feat_body(deg2, pos, seedf, feat, dt, w1a, w1b, w1s, w1c,
               p0a, p0b, p0s, p0c, p0bias, y_out, score0_out):
    deg = deg2[0, :] + deg2[1, :]
    degc = jnp.clip(deg, 0, dt.shape[0] - 1)
    oh = (degc[:, None] == lax.broadcasted_iota(jnp.int32, (1, dt.shape[0]), 1)
          ).astype(jnp.float32)
    demb = jnp.dot(oh, dt[...], preferred_element_type=jnp.float32)
    y = (jnp.dot(pos[...], w1a[...], preferred_element_type=jnp.float32)
         + jnp.dot(demb, w1b[...], preferred_element_type=jnp.float32)
         + seedf[...] * w1s[...]
         + jnp.dot(feat[...], w1c[...], preferred_element_type=jnp.float32))
    y_out[...] = y
    s_pos = jnp.sum(pos[...], axis=0, keepdims=True)
    s_demb = jnp.sum(demb, axis=0, keepdims=True)
    s_seed = jnp.sum(seedf[...], axis=0, keepdims=True)
    s_feat = jnp.sum(feat[...], axis=0, keepdims=True)
    score0_out[...] = (
        jnp.dot(s_pos, p0a[...], preferred_element_type=jnp.float32)
        + jnp.dot(s_demb, p0b[...], preferred_element_type=jnp.float32)
        + s_seed * p0s[...]
        + jnp.dot(s_feat, p0c[...], preferred_element_type=jnp.float32)
        + p0bias[...])


def _finish_body(y, agg2, b1, w2, b2, p1w, p1b, score0, out):
    h = jnp.maximum(y[...] + agg2[0] + agg2[1] + b1[...], 0.0)
    h2 = jnp.maximum(
        jnp.dot(h, w2[...], preferred_element_type=jnp.float32) + b2[...], 0.0)
    pooled1 = jnp.sum(h2, axis=0, keepdims=True)
    out[...] = (score0[...]
                + jnp.dot(pooled1, p1w[...], preferred_element_type=jnp.float32)
                + p1b[...])


@jax.jit
def kernel(edge_index, feat, pos_undirected, seed, deg_table,
           W1, b1, W2, b2, P0_W, P0_b, P1_W, P1_b):
    pos_w = pos_undirected.shape[1]
    demb_w = deg_table.shape[1]
    seedf = seed.astype(jnp.float32)[:, None]

    w1a = W1[:pos_w]
    w1b = W1[pos_w:pos_w + demb_w]
    w1s = W1[pos_w + demb_w:pos_w + demb_w + 1]
    w1c = W1[pos_w + demb_w + 1:]
    p0a = P0_W[:pos_w]
    p0b = P0_W[pos_w:pos_w + demb_w]
    p0s = P0_W[pos_w + demb_w:pos_w + demb_w + 1]
    p0c = P0_W[pos_w + demb_w + 1:]

    src_idx = edge_index[0]
    dst_idx = edge_index[1]
    deg2 = _make_deg_kernel()(src_idx).reshape(NC, N)

    y, score0 = pl.pallas_call(
        _feat_body,
        out_shape=[
            jax.ShapeDtypeStruct((N, HID), jnp.float32),
            jax.ShapeDtypeStruct((1, HID), jnp.float32),
        ],
    )(deg2, pos_undirected, seedf, feat, deg_table,
      w1a, w1b, w1s, w1c, p0a, p0b, p0s, p0c, P0_b[None, :])

    agg2 = _make_agg_kernel()(src_idx, dst_idx, y).reshape(NC, N, HID)

    score = pl.pallas_call(
        _finish_body,
        out_shape=jax.ShapeDtypeStruct((1, HID), jnp.float32),
    )(y, agg2, b1[None, :], W2, b2[None, :], P1_W, P1_b[None, :], score0)
    return score


# speedup vs baseline: 10.8123x; 1.2046x over previous
import functools

import jax
import jax.numpy as jnp
from jax import lax
from jax.experimental import pallas as pl
from jax.experimental.pallas import tpu as pltpu
from jax.experimental.pallas import tpu_sc as plsc

N = 10000
E = 320000
NC = 2
NS = 16
NT = NC * NS
CH = 128
NCHT = E // CH // NT
REM = E // CH - NCHT * NT
HID = 64


def _deg_body(src, deg2, deg_sh, idx_v, rem_v, ones_v, zbuf, sem0, sem1, sem2):
    c = lax.axis_index("c")
    s = lax.axis_index("s")
    w = s * NC + c
    for j in range(8):
        ones_v[pl.ds(j * 16, 16)] = jnp.ones((16,), jnp.int32)
    for j in range(64):
        zbuf[pl.ds(j * 16, 16)] = jnp.zeros((16,), jnp.int32)

    @pl.when(s < 10)
    def _():
        pltpu.sync_copy(zbuf.at[pl.ds(0, 1000)], deg_sh.at[pl.ds(s * 1000, 1000)])

    plsc.subcore_barrier()

    sems = (sem0, sem1, sem2)
    pend = {}

    def load_idx(i, b):
        o = (w + i * NT) * CH
        return pltpu.async_copy(src.at[pl.ds(o, CH)], idx_v.at[b], sems[b])

    pend[0] = load_idx(0, 0)
    pend[1] = load_idx(1, 1)
    for i in range(NCHT):
        b = i % 3
        if i + 2 < NCHT:
            pend[i + 2] = load_idx(i + 2, (i + 2) % 3)
        pend.pop(i).wait()
        pltpu.sync_copy(ones_v, deg_sh.at[idx_v.at[b]], add=True)

    @pl.when(w < REM)
    def _():
        o = (w + NCHT * NT) * CH
        pltpu.sync_copy(src.at[pl.ds(o, CH)], rem_v.at[0])
        pltpu.sync_copy(ones_v, deg_sh.at[rem_v.at[0]], add=True)

    plsc.subcore_barrier()

    @pl.when(s < 10)
    def _():
        pltpu.sync_copy(deg_sh.at[pl.ds(s * 1000, 1000)], zbuf.at[pl.ds(0, 1000)])
        pltpu.sync_copy(zbuf.at[pl.ds(0, 1000)],
                        deg2.at[pl.ds(c * N + s * 1000, 1000)])


@functools.cache
def _make_deg_kernel():
    return pl.kernel(
        _deg_body,
        out_type=jax.ShapeDtypeStruct((NC * N,), jnp.int32),
        mesh=plsc.VectorSubcoreMesh(core_axis_name="c", subcore_axis_name="s"),
        scratch_types=[
            pltpu.VMEM_SHARED((N,), jnp.int32),
            pltpu.VMEM((3, CH), jnp.int32),
            pltpu.VMEM((1, CH), jnp.int32),
            pltpu.VMEM((CH,), jnp.int32),
            pltpu.VMEM((1024,), jnp.int32),
            pltpu.SemaphoreType.DMA,
            pltpu.SemaphoreType.DMA,
            pltpu.SemaphoreType.DMA,
        ],
        compiler_params=pltpu.CompilerParams(use_tc_tiling_on_sc=False),
    )


def _agg_body(src, dst, y, out, agg_sh, src_v, dst_v, rows_v,
              sem0, sem1, sem2):
    c = lax.axis_index("c")
    s = lax.axis_index("s")
    w = s * NC + c

    def zrow(i, carry):
        for j in range(4):
            rows_v[i, pl.ds(j * 16, 16)] = jnp.zeros((16,), jnp.float32)
        return carry

    lax.fori_loop(0, 3 * CH, zrow, 0)

    @pl.when(s < 10)
    def _():
        base = s * 1000
        pltpu.sync_copy(rows_v.at[pl.ds(0, 384)], agg_sh.at[pl.ds(base, 384)])
        pltpu.sync_copy(rows_v.at[pl.ds(0, 384)], agg_sh.at[pl.ds(base + 384, 384)])
        pltpu.sync_copy(rows_v.at[pl.ds(0, 232)], agg_sh.at[pl.ds(base + 768, 232)])

    plsc.subcore_barrier()

    sems = (sem0, sem1, sem2)
    pend = {}

    def stage(i, b):
        o = (w + i * NT) * CH
        pltpu.sync_copy(src.at[pl.ds(o, CH)], src_v.at[b])
        pltpu.sync_copy(dst.at[pl.ds(o, CH)], dst_v.at[b])
        return pltpu.async_copy(y.at[src_v.at[b]],
                                rows_v.at[pl.ds(b * CH, CH)], sems[b])

    pend[0] = stage(0, 0)
    pend[1] = stage(1, 1)
    for i in range(NCHT):
        b = i % 3
        if i + 2 < NCHT:
            pend[i + 2] = stage(i + 2, (i + 2) % 3)
        pend.pop(i).wait()
        pltpu.sync_copy(rows_v.at[pl.ds(b * CH, CH)],
                        agg_sh.at[dst_v.at[b]], add=True)

    @pl.when(w < REM)
    def _():
        stage(NCHT, 0).wait()
        pltpu.sync_copy(rows_v.at[pl.ds(0, CH)],
                        agg_sh.at[dst_v.at[0]], add=True)

    plsc.subcore_barrier()

    @pl.when(s < 10)
    def _():
        for off, nrows in ((0, 384), (384, 384), (768, 232)):
            pltpu.sync_copy(agg_sh.at[pl.ds(s * 1000 + off, nrows)],
                            rows_v.at[pl.ds(0, nrows)])
            pltpu.sync_copy(rows_v.at[pl.ds(0, nrows)],
                            out.at[pl.ds(c * N + s * 1000 + off, nrows), :])


@functools.cache
def _make_agg_kernel():
    return pl.kernel(
        _agg_body,
        out_type=jax.ShapeDtypeStruct((NC * N, HID), jnp.float32),
        mesh=plsc.VectorSubcoreMesh(core_axis_name="c", subcore_axis_name="s"),
        scratch_types=[
            pltpu.VMEM_SHARED((N, HID), jnp.float32),
            pltpu.VMEM((3, CH), jnp.int32),
            pltpu.VMEM((3, CH), jnp.int32),
            pltpu.VMEM((3 * CH, HID), jnp.float32),
            pltpu.SemaphoreType.DMA,
            pltpu.SemaphoreType.DMA,
            pltpu.SemaphoreType.DMA,
        ],
        compiler_params=pltpu.CompilerParams(use_tc_tiling_on_sc=False),
    )


def _feat_body(deg2, pos, seedf, feat, dt, w1a, w1b, w1s, w1c,
               p0a, p0b, p0s, p0c, p0bias, y_out, score0_out):
    deg = deg2[0, :] + deg2[1, :]
    degc = jnp.clip(deg, 0, dt.shape[0] - 1)
    oh = (degc[:, None] == lax.broadcasted_iota(jnp.int32, (1, dt.shape[0]), 1)
          ).astype(jnp.float32)
    demb = jnp.dot(oh, dt[...], preferred_element_type=jnp.float32)
    y = (jnp.dot(pos[...], w1a[...], preferred_element_type=jnp.float32)
         + jnp.dot(demb, w1b[...], preferred_element_type=jnp.float32)
         + seedf[...] * w1s[...]
         + jnp.dot(feat[...], w1c[...], preferred_element_type=jnp.float32))
    y_out[...] = y
    s_pos = jnp.sum(pos[...], axis=0, keepdims=True)
    s_demb = jnp.sum(demb, axis=0, keepdims=True)
    s_seed = jnp.sum(seedf[...], axis=0, keepdims=True)
    s_feat = jnp.sum(feat[...], axis=0, keepdims=True)
    score0_out[...] = (
        jnp.dot(s_pos, p0a[...], preferred_element_type=jnp.float32)
        + jnp.dot(s_demb, p0b[...], preferred_element_type=jnp.float32)
        + s_seed * p0s[...]
        + jnp.dot(s_feat, p0c[...], preferred_element_type=jnp.float32)
        + p0bias[...])


def _finish_body(y, agg2, b1, w2, b2, p1w, p1b, score0, out):
    h = jnp.maximum(y[...] + agg2[0] + agg2[1] + b1[...], 0.0)
    h2 = jnp.maximum(
        jnp.dot(h, w2[...], preferred_element_type=jnp.float32) + b2[...], 0.0)
    pooled1 = jnp.sum(h2, axis=0, keepdims=True)
    out[...] = (score0[...]
                + jnp.dot(pooled1, p1w[...], preferred_element_type=jnp.float32)
                + p1b[...])


@jax.jit
def kernel(edge_index, feat, pos_undirected, seed, deg_table,
           W1, b1, W2, b2, P0_W, P0_b, P1_W, P1_b):
    pos_w = pos_undirected.shape[1]
    demb_w = deg_table.shape[1]
    seedf = seed.astype(jnp.float32)[:, None]

    w1a = W1[:pos_w]
    w1b = W1[pos_w:pos_w + demb_w]
    w1s = W1[pos_w + demb_w:pos_w + demb_w + 1]
    w1c = W1[pos_w + demb_w + 1:]
    p0a = P0_W[:pos_w]
    p0b = P0_W[pos_w:pos_w + demb_w]
    p0s = P0_W[pos_w + demb_w:pos_w + demb_w + 1]
    p0c = P0_W[pos_w + demb_w + 1:]

    src_idx = edge_index[0]
    dst_idx = edge_index[1]
    deg2 = _make_deg_kernel()(src_idx).reshape(NC, N)

    y, score0 = pl.pallas_call(
        _feat_body,
        out_shape=[
            jax.ShapeDtypeStruct((N, HID), jnp.float32),
            jax.ShapeDtypeStruct((1, HID), jnp.float32),
        ],
    )(deg2, pos_undirected, seedf, feat, deg_table,
      w1a, w1b, w1s, w1c, p0a, p0b, p0s, p0c, P0_b[None, :])

    agg2 = _make_agg_kernel()(src_idx, dst_idx, y).reshape(NC, N, HID)

    score = pl.pallas_call(
        _finish_body,
        out_shape=jax.ShapeDtypeStruct((1, HID), jnp.float32),
    )(y, agg2, b1[None, :], W2, b2[None, :], P1_W, P1_b[None, :], score0)
    return score


# speedup vs baseline: 12.2417x; 1.1322x over previous
import functools

import jax
import jax.numpy as jnp
from jax import lax
from jax.experimental import pallas as pl
from jax.experimental.pallas import tpu as pltpu
from jax.experimental.pallas import tpu_sc as plsc

N = 10000
E = 320000
NC = 2
NS = 16
NT = NC * NS
CH = 128
NCHT = E // CH // NT
REM = E // CH - NCHT * NT
HID = 64


def _deg_body(src, deg2, deg_sh, idx_v, rem_v, ones_v, zbuf, sem0, sem1, sem2):
    c = lax.axis_index("c")
    s = lax.axis_index("s")
    w = s * NC + c
    for j in range(8):
        ones_v[pl.ds(j * 16, 16)] = jnp.ones((16,), jnp.int32)
    for j in range(64):
        zbuf[pl.ds(j * 16, 16)] = jnp.zeros((16,), jnp.int32)

    @pl.when(s < 10)
    def _():
        pltpu.sync_copy(zbuf.at[pl.ds(0, 1000)], deg_sh.at[pl.ds(s * 1000, 1000)])

    plsc.subcore_barrier()

    sems = (sem0, sem1, sem2)
    pend = {}

    def load_idx(i, b):
        o = (w + i * NT) * CH
        return pltpu.async_copy(src.at[pl.ds(o, CH)], idx_v.at[b], sems[b])

    pend[0] = load_idx(0, 0)
    pend[1] = load_idx(1, 1)
    for i in range(NCHT):
        b = i % 3
        if i + 2 < NCHT:
            pend[i + 2] = load_idx(i + 2, (i + 2) % 3)
        pend.pop(i).wait()
        pltpu.sync_copy(ones_v, deg_sh.at[idx_v.at[b]], add=True)

    @pl.when(w < REM)
    def _():
        o = (w + NCHT * NT) * CH
        pltpu.sync_copy(src.at[pl.ds(o, CH)], rem_v.at[0])
        pltpu.sync_copy(ones_v, deg_sh.at[rem_v.at[0]], add=True)

    plsc.subcore_barrier()

    @pl.when(s < 10)
    def _():
        pltpu.sync_copy(deg_sh.at[pl.ds(s * 1000, 1000)], zbuf.at[pl.ds(0, 1000)])
        pltpu.sync_copy(zbuf.at[pl.ds(0, 1000)],
                        deg2.at[pl.ds(c * N + s * 1000, 1000)])


@functools.cache
def _make_deg_kernel():
    return pl.kernel(
        _deg_body,
        out_type=jax.ShapeDtypeStruct((NC * N,), jnp.int32),
        mesh=plsc.VectorSubcoreMesh(core_axis_name="c", subcore_axis_name="s"),
        scratch_types=[
            pltpu.VMEM_SHARED((N,), jnp.int32),
            pltpu.VMEM((3, CH), jnp.int32),
            pltpu.VMEM((1, CH), jnp.int32),
            pltpu.VMEM((CH,), jnp.int32),
            pltpu.VMEM((1024,), jnp.int32),
            pltpu.SemaphoreType.DMA,
            pltpu.SemaphoreType.DMA,
            pltpu.SemaphoreType.DMA,
        ],
        compiler_params=pltpu.CompilerParams(use_tc_tiling_on_sc=False),
    )


def _agg_body(src, dst, y, out, agg_sh, src_v, dst_v, rows_v,
              sem0, sem1, sem2, sem3, sem4, sem5):
    c = lax.axis_index("c")
    s = lax.axis_index("s")
    w = s * NC + c

    def zrow(i, carry):
        for j in range(4):
            rows_v[i, pl.ds(j * 16, 16)] = jnp.zeros((16,), jnp.float32)
        return carry

    lax.fori_loop(0, 3 * CH, zrow, 0)

    @pl.when(s < 10)
    def _():
        base = s * 1000
        pltpu.sync_copy(rows_v.at[pl.ds(0, 384)], agg_sh.at[pl.ds(base, 384)])
        pltpu.sync_copy(rows_v.at[pl.ds(0, 384)], agg_sh.at[pl.ds(base + 384, 384)])
        pltpu.sync_copy(rows_v.at[pl.ds(0, 232)], agg_sh.at[pl.ds(base + 768, 232)])

    plsc.subcore_barrier()

    sems = (sem0, sem1, sem2, sem3, sem4, sem5)
    S = 6
    pend_g = {}
    pend_s = {}

    def stage(i):
        b = i % S
        o = (w + i * NT) * CH
        pltpu.sync_copy(src.at[pl.ds(o, CH)], src_v.at[b])
        pltpu.sync_copy(dst.at[pl.ds(o, CH)], dst_v.at[b])
        pend_g[i] = pltpu.async_copy(y.at[src_v.at[b]],
                                     rows_v.at[pl.ds(b * CH, CH)], sems[b])

    stage(0)
    stage(1)
    stage(2)
    for i in range(NCHT):
        b = i % S
        if i + 3 < NCHT:
            if i + 3 >= S:
                pend_s.pop(i + 3 - S).wait()
            stage(i + 3)
        pend_g.pop(i).wait()
        pend_s[i] = pltpu.async_copy(rows_v.at[pl.ds(b * CH, CH)],
                                     agg_sh.at[dst_v.at[b]], sems[b],
                                     add=True)
    for k in sorted(pend_s):
        pend_s[k].wait()

    @pl.when(w < REM)
    def _():
        stage(NCHT)
        pend_g.pop(NCHT).wait()
        pltpu.sync_copy(rows_v.at[pl.ds(0, CH)],
                        agg_sh.at[dst_v.at[0]], add=True)

    plsc.subcore_barrier()

    @pl.when(s < 10)
    def _():
        for off, nrows in ((0, 384), (384, 384), (768, 232)):
            pltpu.sync_copy(agg_sh.at[pl.ds(s * 1000 + off, nrows)],
                            rows_v.at[pl.ds(0, nrows)])
            pltpu.sync_copy(rows_v.at[pl.ds(0, nrows)],
                            out.at[pl.ds(c * N + s * 1000 + off, nrows), :])


@functools.cache
def _make_agg_kernel():
    return pl.kernel(
        _agg_body,
        out_type=jax.ShapeDtypeStruct((NC * N, HID), jnp.float32),
        mesh=plsc.VectorSubcoreMesh(core_axis_name="c", subcore_axis_name="s"),
        scratch_types=[
            pltpu.VMEM_SHARED((N, HID), jnp.float32),
            pltpu.VMEM((6, CH), jnp.int32),
            pltpu.VMEM((6, CH), jnp.int32),
            pltpu.VMEM((6 * CH, HID), jnp.float32),
            pltpu.SemaphoreType.DMA,
            pltpu.SemaphoreType.DMA,
            pltpu.SemaphoreType.DMA,
            pltpu.SemaphoreType.DMA,
            pltpu.SemaphoreType.DMA,
            pltpu.SemaphoreType.DMA,
        ],
        compiler_params=pltpu.CompilerParams(use_tc_tiling_on_sc=False),
    )


def _feat_body(deg2, pos, seedf, feat, dt, w1a, w1b, w1s, w1c,
               p0a, p0b, p0s, p0c, p0bias, y_out, score0_out):
    deg = deg2[0, :] + deg2[1, :]
    degc = jnp.clip(deg, 0, dt.shape[0] - 1)
    oh = (degc[:, None] == lax.broadcasted_iota(jnp.int32, (1, dt.shape[0]), 1)
          ).astype(jnp.float32)
    demb = jnp.dot(oh, dt[...], preferred_element_type=jnp.float32)
    y = (jnp.dot(pos[...], w1a[...], preferred_element_type=jnp.float32)
         + jnp.dot(demb, w1b[...], preferred_element_type=jnp.float32)
         + seedf[...] * w1s[...]
         + jnp.dot(feat[...], w1c[...], preferred_element_type=jnp.float32))
    y_out[...] = y
    s_pos = jnp.sum(pos[...], axis=0, keepdims=True)
    s_demb = jnp.sum(demb, axis=0, keepdims=True)
    s_seed = jnp.sum(seedf[...], axis=0, keepdims=True)
    s_feat = jnp.sum(feat[...], axis=0, keepdims=True)
    score0_out[...] = (
        jnp.dot(s_pos, p0a[...], preferred_element_type=jnp.float32)
        + jnp.dot(s_demb, p0b[...], preferred_element_type=jnp.float32)
        + s_seed * p0s[...]
        + jnp.dot(s_feat, p0c[...], preferred_element_type=jnp.float32)
        + p0bias[...])


def _finish_body(y, agg2, b1, w2, b2, p1w, p1b, score0, out):
    h = jnp.maximum(y[...] + agg2[0] + agg2[1] + b1[...], 0.0)
    h2 = jnp.maximum(
        jnp.dot(h, w2[...], preferred_element_type=jnp.float32) + b2[...], 0.0)
    pooled1 = jnp.sum(h2, axis=0, keepdims=True)
    out[...] = (score0[...]
                + jnp.dot(pooled1, p1w[...], preferred_element_type=jnp.float32)
                + p1b[...])


@jax.jit
def kernel(edge_index, feat, pos_undirected, seed, deg_table,
           W1, b1, W2, b2, P0_W, P0_b, P1_W, P1_b):
    pos_w = pos_undirected.shape[1]
    demb_w = deg_table.shape[1]
    seedf = seed.astype(jnp.float32)[:, None]

    w1a = W1[:pos_w]
    w1b = W1[pos_w:pos_w + demb_w]
    w1s = W1[pos_w + demb_w:pos_w + demb_w + 1]
    w1c = W1[pos_w + demb_w + 1:]
    p0a = P0_W[:pos_w]
    p0b = P0_W[pos_w:pos_w + demb_w]
    p0s = P0_W[pos_w + demb_w:pos_w + demb_w + 1]
    p0c = P0_W[pos_w + demb_w + 1:]

    src_idx = edge_index[0]
    dst_idx = edge_index[1]
    deg2 = _make_deg_kernel()(src_idx).reshape(NC, N)

    y, score0 = pl.pallas_call(
        _feat_body,
        out_shape=[
            jax.ShapeDtypeStruct((N, HID), jnp.float32),
            jax.ShapeDtypeStruct((1, HID), jnp.float32),
        ],
    )(deg2, pos_undirected, seedf, feat, deg_table,
      w1a, w1b, w1s, w1c, p0a, p0b, p0s, p0c, P0_b[None, :])

    agg2 = _make_agg_kernel()(src_idx, dst_idx, y).reshape(NC, N, HID)

    score = pl.pallas_call(
        _finish_body,
        out_shape=jax.ShapeDtypeStruct((1, HID), jnp.float32),
    )(y, agg2, b1[None, :], W2, b2[None, :], P1_W, P1_b[None, :], score0)
    return score


# speedup vs baseline: 15.3754x; 1.2560x over previous
import functools

import jax
import jax.numpy as jnp
from jax import lax
from jax.experimental import pallas as pl
from jax.experimental.pallas import tpu as pltpu
from jax.experimental.pallas import tpu_sc as plsc

N = 10000
E = 320000
NC = 2
NS = 16
NT = NC * NS
CH = 128
NCHT = E // CH // NT
REM = E // CH - NCHT * NT
HID = 64


def _deg_body(src, deg2, deg_sh, idx_v, rem_v, ones_v, zbuf, sem0, sem1, sem2):
    c = lax.axis_index("c")
    s = lax.axis_index("s")
    w = s * NC + c
    for j in range(8):
        ones_v[pl.ds(j * 16, 16)] = jnp.ones((16,), jnp.int32)
    for j in range(64):
        zbuf[pl.ds(j * 16, 16)] = jnp.zeros((16,), jnp.int32)

    @pl.when(s < 10)
    def _():
        pltpu.sync_copy(zbuf.at[pl.ds(0, 1000)], deg_sh.at[pl.ds(s * 1000, 1000)])

    plsc.subcore_barrier()

    sems = (sem0, sem1, sem2)
    pend = {}

    def load_idx(i, b):
        o = (w + i * NT) * CH
        return pltpu.async_copy(src.at[pl.ds(o, CH)], idx_v.at[b], sems[b])

    pend[0] = load_idx(0, 0)
    pend[1] = load_idx(1, 1)
    for i in range(NCHT):
        b = i % 3
        if i + 2 < NCHT:
            pend[i + 2] = load_idx(i + 2, (i + 2) % 3)
        pend.pop(i).wait()
        pltpu.sync_copy(ones_v, deg_sh.at[idx_v.at[b]], add=True)

    @pl.when(w < REM)
    def _():
        o = (w + NCHT * NT) * CH
        pltpu.sync_copy(src.at[pl.ds(o, CH)], rem_v.at[0])
        pltpu.sync_copy(ones_v, deg_sh.at[rem_v.at[0]], add=True)

    plsc.subcore_barrier()

    @pl.when(s < 10)
    def _():
        pltpu.sync_copy(deg_sh.at[pl.ds(s * 1000, 1000)], zbuf.at[pl.ds(0, 1000)])
        pltpu.sync_copy(zbuf.at[pl.ds(0, 1000)],
                        deg2.at[pl.ds(c * N + s * 1000, 1000)])


@functools.cache
def _make_deg_kernel():
    return pl.kernel(
        _deg_body,
        out_type=jax.ShapeDtypeStruct((NC * N,), jnp.int32),
        mesh=plsc.VectorSubcoreMesh(core_axis_name="c", subcore_axis_name="s"),
        scratch_types=[
            pltpu.VMEM_SHARED((N,), jnp.int32),
            pltpu.VMEM((3, CH), jnp.int32),
            pltpu.VMEM((1, CH), jnp.int32),
            pltpu.VMEM((CH,), jnp.int32),
            pltpu.VMEM((1024,), jnp.int32),
            pltpu.SemaphoreType.DMA,
            pltpu.SemaphoreType.DMA,
            pltpu.SemaphoreType.DMA,
        ],
        compiler_params=pltpu.CompilerParams(use_tc_tiling_on_sc=False),
    )


def _agg_body(src, dst, y, out, agg_sh, src_v, dst_v, rows_v,
              sem0, sem1, sem2, sem3, sem4, sem5):
    c = lax.axis_index("c")
    s = lax.axis_index("s")
    w = s * NC + c

    def zrow(i, carry):
        for j in range(4):
            rows_v[i, pl.ds(j * 16, 16)] = jnp.zeros((16,), jnp.float32)
        return carry

    lax.fori_loop(0, 3 * CH, zrow, 0)

    @pl.when(s < 10)
    def _():
        base = s * 1000
        pltpu.sync_copy(rows_v.at[pl.ds(0, 384)], agg_sh.at[pl.ds(base, 384)])
        pltpu.sync_copy(rows_v.at[pl.ds(0, 384)], agg_sh.at[pl.ds(base + 384, 384)])
        pltpu.sync_copy(rows_v.at[pl.ds(0, 232)], agg_sh.at[pl.ds(base + 768, 232)])

    plsc.subcore_barrier()

    sems = (sem0, sem1, sem2, sem3, sem4, sem5)
    S = 6
    pend_i = {}
    pend_g = {}
    pend_s = {}

    def stage_idx(i):
        b = i % S
        o = (w + i * NT) * CH
        pend_i[i] = (
            pltpu.async_copy(src.at[pl.ds(o, CH)], src_v.at[b], sems[b]),
            pltpu.async_copy(dst.at[pl.ds(o, CH)], dst_v.at[b], sems[b]))

    def issue_gath(i):
        b = i % S
        cps, cpd = pend_i.pop(i)
        cps.wait()
        cpd.wait()
        pend_g[i] = pltpu.async_copy(y.at[src_v.at[b]],
                                     rows_v.at[pl.ds(b * CH, CH)], sems[b])

    stage_idx(0)
    stage_idx(1)
    stage_idx(2)
    issue_gath(0)
    issue_gath(1)
    for i in range(NCHT):
        b = i % S
        if i + 3 < NCHT:
            if i - 3 >= 0:
                pend_s.pop(i - 3).wait()
            stage_idx(i + 3)
        if i + 2 < NCHT:
            issue_gath(i + 2)
        pend_g.pop(i).wait()
        pend_s[i] = pltpu.async_copy(rows_v.at[pl.ds(b * CH, CH)],
                                     agg_sh.at[dst_v.at[b]], sems[b],
                                     add=True)
    for k in sorted(pend_s):
        pend_s[k].wait()

    @pl.when(w < REM)
    def _():
        stage_idx(NCHT)
        issue_gath(NCHT)
        pend_g.pop(NCHT).wait()
        pltpu.sync_copy(rows_v.at[pl.ds(0, CH)],
                        agg_sh.at[dst_v.at[0]], add=True)

    plsc.subcore_barrier()

    @pl.when(s < 10)
    def _():
        for off, nrows in ((0, 384), (384, 384), (768, 232)):
            pltpu.sync_copy(agg_sh.at[pl.ds(s * 1000 + off, nrows)],
                            rows_v.at[pl.ds(0, nrows)])
            pltpu.sync_copy(rows_v.at[pl.ds(0, nrows)],
                            out.at[pl.ds(c * N + s * 1000 + off, nrows), :])


@functools.cache
def _make_agg_kernel():
    return pl.kernel(
        _agg_body,
        out_type=jax.ShapeDtypeStruct((NC * N, HID), jnp.float32),
        mesh=plsc.VectorSubcoreMesh(core_axis_name="c", subcore_axis_name="s"),
        scratch_types=[
            pltpu.VMEM_SHARED((N, HID), jnp.float32),
            pltpu.VMEM((6, CH), jnp.int32),
            pltpu.VMEM((6, CH), jnp.int32),
            pltpu.VMEM((6 * CH, HID), jnp.float32),
            pltpu.SemaphoreType.DMA,
            pltpu.SemaphoreType.DMA,
            pltpu.SemaphoreType.DMA,
            pltpu.SemaphoreType.DMA,
            pltpu.SemaphoreType.DMA,
            pltpu.SemaphoreType.DMA,
        ],
        compiler_params=pltpu.CompilerParams(use_tc_tiling_on_sc=False),
    )


def _feat_body(deg2, pos, seedf, feat, dt, w1a, w1b, w1s, w1c,
               p0a, p0b, p0s, p0c, p0bias, y_out, score0_out):
    deg = deg2[0, :] + deg2[1, :]
    degc = jnp.clip(deg, 0, dt.shape[0] - 1)
    oh = (degc[:, None] == lax.broadcasted_iota(jnp.int32, (1, dt.shape[0]), 1)
          ).astype(jnp.float32)
    demb = jnp.dot(oh, dt[...], preferred_element_type=jnp.float32)
    y = (jnp.dot(pos[...], w1a[...], preferred_element_type=jnp.float32)
         + jnp.dot(demb, w1b[...], preferred_element_type=jnp.float32)
         + seedf[...] * w1s[...]
         + jnp.dot(feat[...], w1c[...], preferred_element_type=jnp.float32))
    y_out[...] = y
    s_pos = jnp.sum(pos[...], axis=0, keepdims=True)
    s_demb = jnp.sum(demb, axis=0, keepdims=True)
    s_seed = jnp.sum(seedf[...], axis=0, keepdims=True)
    s_feat = jnp.sum(feat[...], axis=0, keepdims=True)
    score0_out[...] = (
        jnp.dot(s_pos, p0a[...], preferred_element_type=jnp.float32)
        + jnp.dot(s_demb, p0b[...], preferred_element_type=jnp.float32)
        + s_seed * p0s[...]
        + jnp.dot(s_feat, p0c[...], preferred_element_type=jnp.float32)
        + p0bias[...])


def _finish_body(y, agg2, b1, w2, b2, p1w, p1b, score0, out):
    h = jnp.maximum(y[...] + agg2[0] + agg2[1] + b1[...], 0.0)
    h2 = jnp.maximum(
        jnp.dot(h, w2[...], preferred_element_type=jnp.float32) + b2[...], 0.0)
    pooled1 = jnp.sum(h2, axis=0, keepdims=True)
    out[...] = (score0[...]
                + jnp.dot(pooled1, p1w[...], preferred_element_type=jnp.float32)
                + p1b[...])


@jax.jit
def kernel(edge_index, feat, pos_undirected, seed, deg_table,
           W1, b1, W2, b2, P0_W, P0_b, P1_W, P1_b):
    pos_w = pos_undirected.shape[1]
    demb_w = deg_table.shape[1]
    seedf = seed.astype(jnp.float32)[:, None]

    w1a = W1[:pos_w]
    w1b = W1[pos_w:pos_w + demb_w]
    w1s = W1[pos_w + demb_w:pos_w + demb_w + 1]
    w1c = W1[pos_w + demb_w + 1:]
    p0a = P0_W[:pos_w]
    p0b = P0_W[pos_w:pos_w + demb_w]
    p0s = P0_W[pos_w + demb_w:pos_w + demb_w + 1]
    p0c = P0_W[pos_w + demb_w + 1:]

    src_idx = edge_index[0]
    dst_idx = edge_index[1]
    deg2 = _make_deg_kernel()(src_idx).reshape(NC, N)

    y, score0 = pl.pallas_call(
        _feat_body,
        out_shape=[
            jax.ShapeDtypeStruct((N, HID), jnp.float32),
            jax.ShapeDtypeStruct((1, HID), jnp.float32),
        ],
    )(deg2, pos_undirected, seedf, feat, deg_table,
      w1a, w1b, w1s, w1c, p0a, p0b, p0s, p0c, P0_b[None, :])

    agg2 = _make_agg_kernel()(src_idx, dst_idx, y).reshape(NC, N, HID)

    score = pl.pallas_call(
        _finish_body,
        out_shape=jax.ShapeDtypeStruct((1, HID), jnp.float32),
    )(y, agg2, b1[None, :], W2, b2[None, :], P1_W, P1_b[None, :], score0)
    return score


# speedup vs baseline: 15.8079x; 1.0281x over previous
import functools

import jax
import jax.numpy as jnp
from jax import lax
from jax.experimental import pallas as pl
from jax.experimental.pallas import tpu as pltpu
from jax.experimental.pallas import tpu_sc as plsc

N = 10000
E = 320000
NC = 2
NS = 16
NT = NC * NS
CH = 128
NCHT = E // CH // NT
REM = E // CH - NCHT * NT
HID = 64


def _deg_body(src, deg2, deg_sh, idx_v, rem_v, ones_v, zbuf,
              sem0, sem1, sem2, sem3, sem4, sem5):
    c = lax.axis_index("c")
    s = lax.axis_index("s")
    w = s * NC + c
    for j in range(8):
        ones_v[pl.ds(j * 16, 16)] = jnp.ones((16,), jnp.int32)
    for j in range(64):
        zbuf[pl.ds(j * 16, 16)] = jnp.zeros((16,), jnp.int32)

    @pl.when(s < 10)
    def _():
        pltpu.sync_copy(zbuf.at[pl.ds(0, 1000)], deg_sh.at[pl.ds(s * 1000, 1000)])

    plsc.subcore_barrier()

    sems = (sem0, sem1, sem2, sem3, sem4, sem5)
    S = 6
    pend_i = {}
    pend_s = {}

    def load_idx(i):
        b = i % S
        o = (w + i * NT) * CH
        pend_i[i] = pltpu.async_copy(src.at[pl.ds(o, CH)], idx_v.at[b],
                                     sems[b])

    load_idx(0)
    load_idx(1)
    load_idx(2)
    for i in range(NCHT):
        b = i % S
        if i + 3 < NCHT:
            if i - 3 >= 0:
                pend_s.pop(i - 3).wait()
            load_idx(i + 3)
        pend_i.pop(i).wait()
        pend_s[i] = pltpu.async_copy(ones_v, deg_sh.at[idx_v.at[b]],
                                     sems[b], add=True)
    for k in sorted(pend_s):
        pend_s[k].wait()

    @pl.when(w < REM)
    def _():
        o = (w + NCHT * NT) * CH
        pltpu.sync_copy(src.at[pl.ds(o, CH)], rem_v.at[0])
        pltpu.sync_copy(ones_v, deg_sh.at[rem_v.at[0]], add=True)

    plsc.subcore_barrier()

    @pl.when(s < 10)
    def _():
        pltpu.sync_copy(deg_sh.at[pl.ds(s * 1000, 1000)], zbuf.at[pl.ds(0, 1000)])
        pltpu.sync_copy(zbuf.at[pl.ds(0, 1000)],
                        deg2.at[pl.ds(c * N + s * 1000, 1000)])


@functools.cache
def _make_deg_kernel():
    return pl.kernel(
        _deg_body,
        out_type=jax.ShapeDtypeStruct((NC * N,), jnp.int32),
        mesh=plsc.VectorSubcoreMesh(core_axis_name="c", subcore_axis_name="s"),
        scratch_types=[
            pltpu.VMEM_SHARED((N,), jnp.int32),
            pltpu.VMEM((6, CH), jnp.int32),
            pltpu.VMEM((1, CH), jnp.int32),
            pltpu.VMEM((CH,), jnp.int32),
            pltpu.VMEM((1024,), jnp.int32),
            pltpu.SemaphoreType.DMA,
            pltpu.SemaphoreType.DMA,
            pltpu.SemaphoreType.DMA,
            pltpu.SemaphoreType.DMA,
            pltpu.SemaphoreType.DMA,
            pltpu.SemaphoreType.DMA,
        ],
        compiler_params=pltpu.CompilerParams(use_tc_tiling_on_sc=False),
    )


def _agg_body(src, dst, y, out, agg_sh, src_v, dst_v, rows_v,
              sem0, sem1, sem2, sem3, sem4, sem5):
    c = lax.axis_index("c")
    s = lax.axis_index("s")
    w = s * NC + c

    def zrow(i, carry):
        for j in range(4):
            rows_v[i, pl.ds(j * 16, 16)] = jnp.zeros((16,), jnp.float32)
        return carry

    lax.fori_loop(0, 3 * CH, zrow, 0)

    @pl.when(s < 10)
    def _():
        base = s * 1000
        pltpu.sync_copy(rows_v.at[pl.ds(0, 384)], agg_sh.at[pl.ds(base, 384)])
        pltpu.sync_copy(rows_v.at[pl.ds(0, 384)], agg_sh.at[pl.ds(base + 384, 384)])
        pltpu.sync_copy(rows_v.at[pl.ds(0, 232)], agg_sh.at[pl.ds(base + 768, 232)])

    plsc.subcore_barrier()

    sems = (sem0, sem1, sem2, sem3, sem4, sem5)
    S = 6
    pend_i = {}
    pend_g = {}
    pend_s = {}

    def stage_idx(i):
        b = i % S
        o = (w + i * NT) * CH
        pend_i[i] = (
            pltpu.async_copy(src.at[pl.ds(o, CH)], src_v.at[b], sems[b]),
            pltpu.async_copy(dst.at[pl.ds(o, CH)], dst_v.at[b], sems[b]))

    def issue_gath(i):
        b = i % S
        cps, cpd = pend_i.pop(i)
        cps.wait()
        cpd.wait()
        pend_g[i] = pltpu.async_copy(y.at[src_v.at[b]],
                                     rows_v.at[pl.ds(b * CH, CH)], sems[b])

    stage_idx(0)
    stage_idx(1)
    stage_idx(2)
    issue_gath(0)
    issue_gath(1)
    for i in range(NCHT):
        b = i % S
        if i + 3 < NCHT:
            if i - 3 >= 0:
                pend_s.pop(i - 3).wait()
            stage_idx(i + 3)
        if i + 2 < NCHT:
            issue_gath(i + 2)
        pend_g.pop(i).wait()
        pend_s[i] = pltpu.async_copy(rows_v.at[pl.ds(b * CH, CH)],
                                     agg_sh.at[dst_v.at[b]], sems[b],
                                     add=True)
    for k in sorted(pend_s):
        pend_s[k].wait()

    @pl.when(w < REM)
    def _():
        stage_idx(NCHT)
        issue_gath(NCHT)
        pend_g.pop(NCHT).wait()
        pltpu.sync_copy(rows_v.at[pl.ds(0, CH)],
                        agg_sh.at[dst_v.at[0]], add=True)

    plsc.subcore_barrier()

    @pl.when(s < 10)
    def _():
        for off, nrows in ((0, 384), (384, 384), (768, 232)):
            pltpu.sync_copy(agg_sh.at[pl.ds(s * 1000 + off, nrows)],
                            rows_v.at[pl.ds(0, nrows)])
            pltpu.sync_copy(rows_v.at[pl.ds(0, nrows)],
                            out.at[pl.ds(c * N + s * 1000 + off, nrows), :])


@functools.cache
def _make_agg_kernel():
    return pl.kernel(
        _agg_body,
        out_type=jax.ShapeDtypeStruct((NC * N, HID), jnp.float32),
        mesh=plsc.VectorSubcoreMesh(core_axis_name="c", subcore_axis_name="s"),
        scratch_types=[
            pltpu.VMEM_SHARED((N, HID), jnp.float32),
            pltpu.VMEM((6, CH), jnp.int32),
            pltpu.VMEM((6, CH), jnp.int32),
            pltpu.VMEM((6 * CH, HID), jnp.float32),
            pltpu.SemaphoreType.DMA,
            pltpu.SemaphoreType.DMA,
            pltpu.SemaphoreType.DMA,
            pltpu.SemaphoreType.DMA,
            pltpu.SemaphoreType.DMA,
            pltpu.SemaphoreType.DMA,
        ],
        compiler_params=pltpu.CompilerParams(use_tc_tiling_on_sc=False),
    )


def _feat_body(deg2, pos, seedf, feat, dt, w1a, w1b, w1s, w1c,
               p0a, p0b, p0s, p0c, p0bias, y_out, score0_out):
    deg = deg2[0, :] + deg2[1, :]
    degc = jnp.clip(deg, 0, dt.shape[0] - 1)
    oh = (degc[:, None] == lax.broadcasted_iota(jnp.int32, (1, dt.shape[0]), 1)
          ).astype(jnp.float32)
    demb = jnp.dot(oh, dt[...], preferred_element_type=jnp.float32)
    y = (jnp.dot(pos[...], w1a[...], preferred_element_type=jnp.float32)
         + jnp.dot(demb, w1b[...], preferred_element_type=jnp.float32)
         + seedf[...] * w1s[...]
         + jnp.dot(feat[...], w1c[...], preferred_element_type=jnp.float32))
    y_out[...] = y
    s_pos = jnp.sum(pos[...], axis=0, keepdims=True)
    s_demb = jnp.sum(demb, axis=0, keepdims=True)
    s_seed = jnp.sum(seedf[...], axis=0, keepdims=True)
    s_feat = jnp.sum(feat[...], axis=0, keepdims=True)
    score0_out[...] = (
        jnp.dot(s_pos, p0a[...], preferred_element_type=jnp.float32)
        + jnp.dot(s_demb, p0b[...], preferred_element_type=jnp.float32)
        + s_seed * p0s[...]
        + jnp.dot(s_feat, p0c[...], preferred_element_type=jnp.float32)
        + p0bias[...])


def _finish_body(y, agg2, b1, w2, b2, p1w, p1b, score0, out):
    h = jnp.maximum(y[...] + agg2[0] + agg2[1] + b1[...], 0.0)
    h2 = jnp.maximum(
        jnp.dot(h, w2[...], preferred_element_type=jnp.float32) + b2[...], 0.0)
    pooled1 = jnp.sum(h2, axis=0, keepdims=True)
    out[...] = (score0[...]
                + jnp.dot(pooled1, p1w[...], preferred_element_type=jnp.float32)
                + p1b[...])


@jax.jit
def kernel(edge_index, feat, pos_undirected, seed, deg_table,
           W1, b1, W2, b2, P0_W, P0_b, P1_W, P1_b):
    pos_w = pos_undirected.shape[1]
    demb_w = deg_table.shape[1]
    seedf = seed.astype(jnp.float32)[:, None]

    w1a = W1[:pos_w]
    w1b = W1[pos_w:pos_w + demb_w]
    w1s = W1[pos_w + demb_w:pos_w + demb_w + 1]
    w1c = W1[pos_w + demb_w + 1:]
    p0a = P0_W[:pos_w]
    p0b = P0_W[pos_w:pos_w + demb_w]
    p0s = P0_W[pos_w + demb_w:pos_w + demb_w + 1]
    p0c = P0_W[pos_w + demb_w + 1:]

    src_idx = edge_index[0]
    dst_idx = edge_index[1]
    deg2 = _make_deg_kernel()(src_idx).reshape(NC, N)

    y, score0 = pl.pallas_call(
        _feat_body,
        out_shape=[
            jax.ShapeDtypeStruct((N, HID), jnp.float32),
            jax.ShapeDtypeStruct((1, HID), jnp.float32),
        ],
    )(deg2, pos_undirected, seedf, feat, deg_table,
      w1a, w1b, w1s, w1c, p0a, p0b, p0s, p0c, P0_b[None, :])

    agg2 = _make_agg_kernel()(src_idx, dst_idx, y).reshape(NC, N, HID)

    score = pl.pallas_call(
        _finish_body,
        out_shape=jax.ShapeDtypeStruct((1, HID), jnp.float32),
    )(y, agg2, b1[None, :], W2, b2[None, :], P1_W, P1_b[None, :], score0)
    return score


# speedup vs baseline: 16.0435x; 1.0149x over previous
import functools

import jax
import jax.numpy as jnp
from jax import lax
from jax.experimental import pallas as pl
from jax.experimental.pallas import tpu as pltpu
from jax.experimental.pallas import tpu_sc as plsc

N = 10000
E = 320000
NC = 2
NS = 16
NT = NC * NS
CH = 128
NCHT = E // CH // NT
REM = E // CH - NCHT * NT
HID = 64


def _deg_body(src, deg2, deg_sh, idx_v, rem_v, ones_v, zbuf,
              sem0, sem1, sem2, sem3, sem4, sem5):
    c = lax.axis_index("c")
    s = lax.axis_index("s")
    w = s * NC + c
    for j in range(8):
        ones_v[pl.ds(j * 16, 16)] = jnp.ones((16,), jnp.int32)
    for j in range(64):
        zbuf[pl.ds(j * 16, 16)] = jnp.zeros((16,), jnp.int32)

    @pl.when(s < 10)
    def _():
        pltpu.sync_copy(zbuf.at[pl.ds(0, 1000)], deg_sh.at[pl.ds(s * 1000, 1000)])

    plsc.subcore_barrier()

    sems = (sem0, sem1, sem2, sem3, sem4, sem5)
    S = 6
    pend_i = {}
    pend_s = {}

    def load_idx(i):
        b = i % S
        o = (w + i * NT) * CH
        pend_i[i] = pltpu.async_copy(src.at[pl.ds(o, CH)], idx_v.at[b],
                                     sems[b])

    load_idx(0)
    load_idx(1)
    load_idx(2)
    for i in range(NCHT):
        b = i % S
        if i + 3 < NCHT:
            if i - 3 >= 0:
                pend_s.pop(i - 3).wait()
            load_idx(i + 3)
        pend_i.pop(i).wait()
        pend_s[i] = pltpu.async_copy(ones_v, deg_sh.at[idx_v.at[b]],
                                     sems[b], add=True)
    for k in sorted(pend_s):
        pend_s[k].wait()

    @pl.when(w < REM)
    def _():
        o = (w + NCHT * NT) * CH
        pltpu.sync_copy(src.at[pl.ds(o, CH)], rem_v.at[0])
        pltpu.sync_copy(ones_v, deg_sh.at[rem_v.at[0]], add=True)

    plsc.subcore_barrier()

    @pl.when(s < 10)
    def _():
        pltpu.sync_copy(deg_sh.at[pl.ds(s * 1000, 1000)], zbuf.at[pl.ds(0, 1000)])
        pltpu.sync_copy(zbuf.at[pl.ds(0, 1000)],
                        deg2.at[pl.ds(c * N + s * 1000, 1000)])


@functools.cache
def _make_deg_kernel():
    return pl.kernel(
        _deg_body,
        out_type=jax.ShapeDtypeStruct((NC * N,), jnp.int32),
        mesh=plsc.VectorSubcoreMesh(core_axis_name="c", subcore_axis_name="s"),
        scratch_types=[
            pltpu.VMEM_SHARED((N,), jnp.int32),
            pltpu.VMEM((6, CH), jnp.int32),
            pltpu.VMEM((1, CH), jnp.int32),
            pltpu.VMEM((CH,), jnp.int32),
            pltpu.VMEM((1024,), jnp.int32),
            pltpu.SemaphoreType.DMA,
            pltpu.SemaphoreType.DMA,
            pltpu.SemaphoreType.DMA,
            pltpu.SemaphoreType.DMA,
            pltpu.SemaphoreType.DMA,
            pltpu.SemaphoreType.DMA,
        ],
        compiler_params=pltpu.CompilerParams(use_tc_tiling_on_sc=False),
    )


def _agg_body(src, dst, y, out, agg_sh, src_v, dst_v, rows_v,
              sem0, sem1, sem2, sem3, sem4, sem5, sem6, sem7):
    c = lax.axis_index("c")
    s = lax.axis_index("s")
    w = s * NC + c

    def zrow(i, carry):
        for j in range(4):
            rows_v[i, pl.ds(j * 16, 16)] = jnp.zeros((16,), jnp.float32)
        return carry

    lax.fori_loop(0, 3 * CH, zrow, 0)

    @pl.when(s < 10)
    def _():
        base = s * 1000
        pltpu.sync_copy(rows_v.at[pl.ds(0, 384)], agg_sh.at[pl.ds(base, 384)])
        pltpu.sync_copy(rows_v.at[pl.ds(0, 384)], agg_sh.at[pl.ds(base + 384, 384)])
        pltpu.sync_copy(rows_v.at[pl.ds(0, 232)], agg_sh.at[pl.ds(base + 768, 232)])

    plsc.subcore_barrier()

    sems = (sem0, sem1, sem2, sem3, sem4, sem5, sem6, sem7)
    S = 8
    pend_i = {}
    pend_g = {}
    pend_s = {}

    def stage_idx(i):
        b = i % S
        o = (w + i * NT) * CH
        pend_i[i] = (
            pltpu.async_copy(src.at[pl.ds(o, CH)], src_v.at[b], sems[b]),
            pltpu.async_copy(dst.at[pl.ds(o, CH)], dst_v.at[b], sems[b]))

    def issue_gath(i):
        b = i % S
        cps, cpd = pend_i.pop(i)
        cps.wait()
        cpd.wait()
        pend_g[i] = pltpu.async_copy(y.at[src_v.at[b]],
                                     rows_v.at[pl.ds(b * CH, CH)], sems[b])

    for i in range(5):
        stage_idx(i)
    for i in range(3):
        issue_gath(i)
    for i in range(NCHT):
        b = i % S
        if i - 3 >= 0:
            pend_s.pop(i - 3).wait()
        if i + 5 < NCHT:
            stage_idx(i + 5)
        if i + 3 < NCHT:
            issue_gath(i + 3)
        pend_g.pop(i).wait()
        pend_s[i] = pltpu.async_copy(rows_v.at[pl.ds(b * CH, CH)],
                                     agg_sh.at[dst_v.at[b]], sems[b],
                                     add=True)
    for k in sorted(pend_s):
        pend_s[k].wait()

    @pl.when(w < REM)
    def _():
        rb = NCHT % S
        stage_idx(NCHT)
        issue_gath(NCHT)
        pend_g.pop(NCHT).wait()
        pltpu.sync_copy(rows_v.at[pl.ds(rb * CH, CH)],
                        agg_sh.at[dst_v.at[rb]], add=True)

    plsc.subcore_barrier()

    @pl.when(s < 10)
    def _():
        for off, nrows in ((0, 384), (384, 384), (768, 232)):
            pltpu.sync_copy(agg_sh.at[pl.ds(s * 1000 + off, nrows)],
                            rows_v.at[pl.ds(0, nrows)])
            pltpu.sync_copy(rows_v.at[pl.ds(0, nrows)],
                            out.at[pl.ds(c * N + s * 1000 + off, nrows), :])


@functools.cache
def _make_agg_kernel():
    return pl.kernel(
        _agg_body,
        out_type=jax.ShapeDtypeStruct((NC * N, HID), jnp.float32),
        mesh=plsc.VectorSubcoreMesh(core_axis_name="c", subcore_axis_name="s"),
        scratch_types=[
            pltpu.VMEM_SHARED((N, HID), jnp.float32),
            pltpu.VMEM((8, CH), jnp.int32),
            pltpu.VMEM((8, CH), jnp.int32),
            pltpu.VMEM((8 * CH, HID), jnp.float32),
            pltpu.SemaphoreType.DMA,
            pltpu.SemaphoreType.DMA,
            pltpu.SemaphoreType.DMA,
            pltpu.SemaphoreType.DMA,
            pltpu.SemaphoreType.DMA,
            pltpu.SemaphoreType.DMA,
            pltpu.SemaphoreType.DMA,
            pltpu.SemaphoreType.DMA,
        ],
        compiler_params=pltpu.CompilerParams(use_tc_tiling_on_sc=False),
    )


def _feat_body(deg2, pos, seedf, feat, dt, w1a, w1b, w1s, w1c,
               p0a, p0b, p0s, p0c, p0bias, y_out, score0_out):
    deg = deg2[0, :] + deg2[1, :]
    degc = jnp.clip(deg, 0, dt.shape[0] - 1)
    oh = (degc[:, None] == lax.broadcasted_iota(jnp.int32, (1, dt.shape[0]), 1)
          ).astype(jnp.float32)
    demb = jnp.dot(oh, dt[...], preferred_element_type=jnp.float32)
    y = (jnp.dot(pos[...], w1a[...], preferred_element_type=jnp.float32)
         + jnp.dot(demb, w1b[...], preferred_element_type=jnp.float32)
         + seedf[...] * w1s[...]
         + jnp.dot(feat[...], w1c[...], preferred_element_type=jnp.float32))
    y_out[...] = y
    s_pos = jnp.sum(pos[...], axis=0, keepdims=True)
    s_demb = jnp.sum(demb, axis=0, keepdims=True)
    s_seed = jnp.sum(seedf[...], axis=0, keepdims=True)
    s_feat = jnp.sum(feat[...], axis=0, keepdims=True)
    score0_out[...] = (
        jnp.dot(s_pos, p0a[...], preferred_element_type=jnp.float32)
        + jnp.dot(s_demb, p0b[...], preferred_element_type=jnp.float32)
        + s_seed * p0s[...]
        + jnp.dot(s_feat, p0c[...], preferred_element_type=jnp.float32)
        + p0bias[...])


def _finish_body(y, agg2, b1, w2, b2, p1w, p1b, score0, out):
    h = jnp.maximum(y[...] + agg2[0] + agg2[1] + b1[...], 0.0)
    h2 = jnp.maximum(
        jnp.dot(h, w2[...], preferred_element_type=jnp.float32) + b2[...], 0.0)
    pooled1 = jnp.sum(h2, axis=0, keepdims=True)
    out[...] = (score0[...]
                + jnp.dot(pooled1, p1w[...], preferred_element_type=jnp.float32)
                + p1b[...])


@jax.jit
def kernel(edge_index, feat, pos_undirected, seed, deg_table,
           W1, b1, W2, b2, P0_W, P0_b, P1_W, P1_b):
    pos_w = pos_undirected.shape[1]
    demb_w = deg_table.shape[1]
    seedf = seed.astype(jnp.float32)[:, None]

    w1a = W1[:pos_w]
    w1b = W1[pos_w:pos_w + demb_w]
    w1s = W1[pos_w + demb_w:pos_w + demb_w + 1]
    w1c = W1[pos_w + demb_w + 1:]
    p0a = P0_W[:pos_w]
    p0b = P0_W[pos_w:pos_w + demb_w]
    p0s = P0_W[pos_w + demb_w:pos_w + demb_w + 1]
    p0c = P0_W[pos_w + demb_w + 1:]

    src_idx = edge_index[0]
    dst_idx = edge_index[1]
    deg2 = _make_deg_kernel()(src_idx).reshape(NC, N)

    y, score0 = pl.pallas_call(
        _feat_body,
        out_shape=[
            jax.ShapeDtypeStruct((N, HID), jnp.float32),
            jax.ShapeDtypeStruct((1, HID), jnp.float32),
        ],
    )(deg2, pos_undirected, seedf, feat, deg_table,
      w1a, w1b, w1s, w1c, p0a, p0b, p0s, p0c, P0_b[None, :])

    agg2 = _make_agg_kernel()(src_idx, dst_idx, y).reshape(NC, N, HID)

    score = pl.pallas_call(
        _finish_body,
        out_shape=jax.ShapeDtypeStruct((1, HID), jnp.float32),
    )(y, agg2, b1[None, :], W2, b2[None, :], P1_W, P1_b[None, :], score0)
    return score


# speedup vs baseline: 16.0846x; 1.0026x over previous
import functools

import jax
import jax.numpy as jnp
from jax import lax
from jax.experimental import pallas as pl
from jax.experimental.pallas import tpu as pltpu
from jax.experimental.pallas import tpu_sc as plsc

N = 10000
E = 320000
NC = 2
NS = 16
NT = NC * NS
CH = 128
NCHT = E // CH // NT
REM = E // CH - NCHT * NT
HID = 64


def _deg_body(src, deg2, deg_sh, idx_v, rem_v, ones_v, zbuf,
              sem0, sem1, sem2, sem3, sem4, sem5):
    c = lax.axis_index("c")
    s = lax.axis_index("s")
    w = s * NC + c
    for j in range(8):
        ones_v[pl.ds(j * 16, 16)] = jnp.ones((16,), jnp.int32)
    for j in range(64):
        zbuf[pl.ds(j * 16, 16)] = jnp.zeros((16,), jnp.int32)

    @pl.when(s < 10)
    def _():
        pltpu.sync_copy(zbuf.at[pl.ds(0, 1000)], deg_sh.at[pl.ds(s * 1000, 1000)])

    plsc.subcore_barrier()

    sems = (sem0, sem1, sem2, sem3, sem4, sem5)
    S = 6
    pend_i = {}
    pend_s = {}

    def load_idx(i):
        b = i % S
        o = (w + i * NT) * CH
        pend_i[i] = pltpu.async_copy(src.at[pl.ds(o, CH)], idx_v.at[b],
                                     sems[b])

    load_idx(0)
    load_idx(1)
    load_idx(2)
    for i in range(NCHT):
        b = i % S
        if i + 3 < NCHT:
            if i - 3 >= 0:
                pend_s.pop(i - 3).wait()
            load_idx(i + 3)
        pend_i.pop(i).wait()
        pend_s[i] = pltpu.async_copy(ones_v, deg_sh.at[idx_v.at[b]],
                                     sems[b], add=True)
    for k in sorted(pend_s):
        pend_s[k].wait()

    @pl.when(w < REM)
    def _():
        o = (w + NCHT * NT) * CH
        pltpu.sync_copy(src.at[pl.ds(o, CH)], rem_v.at[0])
        pltpu.sync_copy(ones_v, deg_sh.at[rem_v.at[0]], add=True)

    plsc.subcore_barrier()

    @pl.when(s < 10)
    def _():
        pltpu.sync_copy(deg_sh.at[pl.ds(s * 1000, 1000)], zbuf.at[pl.ds(0, 1000)])
        pltpu.sync_copy(zbuf.at[pl.ds(0, 1000)],
                        deg2.at[pl.ds(c * N + s * 1000, 1000)])


@functools.cache
def _make_deg_kernel():
    return pl.kernel(
        _deg_body,
        out_type=jax.ShapeDtypeStruct((NC * N,), jnp.int32),
        mesh=plsc.VectorSubcoreMesh(core_axis_name="c", subcore_axis_name="s"),
        scratch_types=[
            pltpu.VMEM_SHARED((N,), jnp.int32),
            pltpu.VMEM((6, CH), jnp.int32),
            pltpu.VMEM((1, CH), jnp.int32),
            pltpu.VMEM((CH,), jnp.int32),
            pltpu.VMEM((1024,), jnp.int32),
            pltpu.SemaphoreType.DMA,
            pltpu.SemaphoreType.DMA,
            pltpu.SemaphoreType.DMA,
            pltpu.SemaphoreType.DMA,
            pltpu.SemaphoreType.DMA,
            pltpu.SemaphoreType.DMA,
        ],
        compiler_params=pltpu.CompilerParams(use_tc_tiling_on_sc=False),
    )


def _agg_body(src, dst, y, out, agg_sh, src_v, dst_v, rows_v,
              sem0, sem1, sem2, sem3, sem4, sem5, sem6, sem7):
    c = lax.axis_index("c")
    s = lax.axis_index("s")
    w = s * NC + c

    def zrow(i, carry):
        for j in range(4):
            rows_v[i, pl.ds(j * 16, 16)] = jnp.zeros((16,), jnp.float32)
        return carry

    lax.fori_loop(0, 3 * CH, zrow, 0)

    @pl.when(s < 10)
    def _():
        base = s * 1000
        pltpu.sync_copy(rows_v.at[pl.ds(0, 384)], agg_sh.at[pl.ds(base, 384)])
        pltpu.sync_copy(rows_v.at[pl.ds(0, 384)], agg_sh.at[pl.ds(base + 384, 384)])
        pltpu.sync_copy(rows_v.at[pl.ds(0, 232)], agg_sh.at[pl.ds(base + 768, 232)])

    plsc.subcore_barrier()

    sems = (sem0, sem1, sem2, sem3, sem4, sem5, sem6, sem7)
    S = 8
    pend_i = {}
    pend_g = {}
    pend_s = {}

    def stage_idx(i):
        b = i % S
        o = (w + i * NT) * CH
        pend_i[i] = (
            pltpu.async_copy(src.at[pl.ds(o, CH)], src_v.at[b], sems[b]),
            pltpu.async_copy(dst.at[pl.ds(o, CH)], dst_v.at[b], sems[b]))

    def issue_gath(i):
        b = i % S
        cps, cpd = pend_i.pop(i)
        cps.wait()
        cpd.wait()
        pend_g[i] = pltpu.async_copy(y.at[src_v.at[b]],
                                     rows_v.at[pl.ds(b * CH, CH)], sems[b])

    for i in range(5):
        stage_idx(i)
    for i in range(3):
        issue_gath(i)
    for i in range(NCHT):
        b = i % S
        if i - 3 >= 0:
            pend_s.pop(i - 3).wait()
        if i + 5 < NCHT:
            stage_idx(i + 5)
        if i + 3 < NCHT:
            issue_gath(i + 3)
        pend_g.pop(i).wait()
        pend_s[i] = pltpu.async_copy(rows_v.at[pl.ds(b * CH, CH)],
                                     agg_sh.at[dst_v.at[b]], sems[b],
                                     add=True)
    for k in sorted(pend_s):
        pend_s[k].wait()

    @pl.when(w < REM)
    def _():
        rb = NCHT % S
        stage_idx(NCHT)
        issue_gath(NCHT)
        pend_g.pop(NCHT).wait()
        pltpu.sync_copy(rows_v.at[pl.ds(rb * CH, CH)],
                        agg_sh.at[dst_v.at[rb]], add=True)

    plsc.subcore_barrier()

    @pl.when(s < 10)
    def _():
        pltpu.sync_copy(agg_sh.at[pl.ds(s * 1000, 1000)],
                        rows_v.at[pl.ds(0, 1000)])
        pltpu.sync_copy(rows_v.at[pl.ds(0, 1000)],
                        out.at[pl.ds(c * N + s * 1000, 1000), :])


@functools.cache
def _make_agg_kernel():
    return pl.kernel(
        _agg_body,
        out_type=jax.ShapeDtypeStruct((NC * N, HID), jnp.float32),
        mesh=plsc.VectorSubcoreMesh(core_axis_name="c", subcore_axis_name="s"),
        scratch_types=[
            pltpu.VMEM_SHARED((N, HID), jnp.float32),
            pltpu.VMEM((8, CH), jnp.int32),
            pltpu.VMEM((8, CH), jnp.int32),
            pltpu.VMEM((8 * CH, HID), jnp.float32),
            pltpu.SemaphoreType.DMA,
            pltpu.SemaphoreType.DMA,
            pltpu.SemaphoreType.DMA,
            pltpu.SemaphoreType.DMA,
            pltpu.SemaphoreType.DMA,
            pltpu.SemaphoreType.DMA,
            pltpu.SemaphoreType.DMA,
            pltpu.SemaphoreType.DMA,
        ],
        compiler_params=pltpu.CompilerParams(use_tc_tiling_on_sc=False),
    )


def _feat_body(deg2, pos, seedf, feat, dt, w1a, w1b, w1s, w1c,
               p0a, p0b, p0s, p0c, p0bias, y_out, score0_out):
    deg = deg2[0, :] + deg2[1, :]
    degc = jnp.clip(deg, 0, dt.shape[0] - 1)
    oh = (degc[:, None] == lax.broadcasted_iota(jnp.int32, (1, dt.shape[0]), 1)
          ).astype(jnp.float32)
    demb = jnp.dot(oh, dt[...], preferred_element_type=jnp.float32)
    y = (jnp.dot(pos[...], w1a[...], preferred_element_type=jnp.float32)
         + jnp.dot(demb, w1b[...], preferred_element_type=jnp.float32)
         + seedf[...] * w1s[...]
         + jnp.dot(feat[...], w1c[...], preferred_element_type=jnp.float32))
    y_out[...] = y
    s_pos = jnp.sum(pos[...], axis=0, keepdims=True)
    s_demb = jnp.sum(demb, axis=0, keepdims=True)
    s_seed = jnp.sum(seedf[...], axis=0, keepdims=True)
    s_feat = jnp.sum(feat[...], axis=0, keepdims=True)
    score0_out[...] = (
        jnp.dot(s_pos, p0a[...], preferred_element_type=jnp.float32)
        + jnp.dot(s_demb, p0b[...], preferred_element_type=jnp.float32)
        + s_seed * p0s[...]
        + jnp.dot(s_feat, p0c[...], preferred_element_type=jnp.float32)
        + p0bias[...])


def _finish_body(y, agg2, b1, w2, b2, p1w, p1b, score0, out):
    h = jnp.maximum(y[...] + agg2[0] + agg2[1] + b1[...], 0.0)
    h2 = jnp.maximum(
        jnp.dot(h, w2[...], preferred_element_type=jnp.float32) + b2[...], 0.0)
    pooled1 = jnp.sum(h2, axis=0, keepdims=True)
    out[...] = (score0[...]
                + jnp.dot(pooled1, p1w[...], preferred_element_type=jnp.float32)
                + p1b[...])


@jax.jit
def kernel(edge_index, feat, pos_undirected, seed, deg_table,
           W1, b1, W2, b2, P0_W, P0_b, P1_W, P1_b):
    pos_w = pos_undirected.shape[1]
    demb_w = deg_table.shape[1]
    seedf = seed.astype(jnp.float32)[:, None]

    w1a = W1[:pos_w]
    w1b = W1[pos_w:pos_w + demb_w]
    w1s = W1[pos_w + demb_w:pos_w + demb_w + 1]
    w1c = W1[pos_w + demb_w + 1:]
    p0a = P0_W[:pos_w]
    p0b = P0_W[pos_w:pos_w + demb_w]
    p0s = P0_W[pos_w + demb_w:pos_w + demb_w + 1]
    p0c = P0_W[pos_w + demb_w + 1:]

    src_idx = edge_index[0]
    dst_idx = edge_index[1]
    deg2 = _make_deg_kernel()(src_idx).reshape(NC, N)

    y, score0 = pl.pallas_call(
        _feat_body,
        out_shape=[
            jax.ShapeDtypeStruct((N, HID), jnp.float32),
            jax.ShapeDtypeStruct((1, HID), jnp.float32),
        ],
    )(deg2, pos_undirected, seedf, feat, deg_table,
      w1a, w1b, w1s, w1c, p0a, p0b, p0s, p0c, P0_b[None, :])

    agg2 = _make_agg_kernel()(src_idx, dst_idx, y).reshape(NC, N, HID)

    score = pl.pallas_call(
        _finish_body,
        out_shape=jax.ShapeDtypeStruct((1, HID), jnp.float32),
    )(y, agg2, b1[None, :], W2, b2[None, :], P1_W, P1_b[None, :], score0)
    return score
